# Initial kernel scaffold; baseline (speedup 1.0000x reference)
import jax
import jax.numpy as jnp
from jax.experimental import pallas as pl


def kernel(boxes, scores, labels):
    raise NotImplementedError("write your pallas kernel here")


# speedup vs baseline: 589.5217x; 589.5217x over previous
import functools

import jax
import jax.numpy as jnp
import numpy as np
from jax import lax
from jax.experimental import pallas as pl
from jax.experimental.pallas import tpu as pltpu
from jax.experimental.pallas import tpu_sc as plsc

_N = 5000
_NPAD = 5008
_NCHUNK = _NPAD // 16
_CMVECS = 20
_NEGF = np.float32(-np.inf)
_SCORE_THRESH = np.float32(0.2)
_NMS_THRESH = np.float32(0.5)
_MAXK = 15

_mesh = plsc.VectorSubcoreMesh(core_axis_name="c", subcore_axis_name="s")

_f32 = np.float32
_i32 = np.int32


def _iota16():
    return lax.broadcasted_iota(_i32, (16,), 0)


@functools.partial(
    pl.kernel,
    out_type=[jax.ShapeDtypeStruct((32,), _f32)] * 5
    + [jax.ShapeDtypeStruct((32,), _i32)],
    mesh=_mesh,
    compiler_params=pltpu.CompilerParams(needs_layout_passes=False),
    scratch_types=[
        pltpu.VMEM((_NPAD,), _f32),
        pltpu.VMEM((_NPAD,), _f32),
        pltpu.VMEM((_NPAD,), _f32),
        pltpu.VMEM((_NPAD,), _f32),
        pltpu.VMEM((_NPAD,), _f32),
        pltpu.VMEM((_NPAD,), _i32),
        pltpu.VMEM((_NPAD,), _f32),
        pltpu.VMEM((_NPAD,), _f32),
        pltpu.VMEM((_CMVECS * 16,), _f32),
        pltpu.VMEM((_CMVECS * 16,), _f32),
        pltpu.VMEM((32,), _f32),
        pltpu.VMEM((32,), _f32),
        pltpu.VMEM((32,), _f32),
        pltpu.VMEM((32,), _f32),
        pltpu.VMEM((32,), _f32),
        pltpu.VMEM((32,), _i32),
        pltpu.SemaphoreType.DMA,
    ],
)
def _nms_sc(x1_h, y1_h, x2_h, y2_h, sc_h, lb_h,
            ox1_h, oy1_h, ox2_h, oy2_h, osc_h, olb_h,
            x1_v, y1_v, x2_v, y2_v, sc_v, lb_v,
            effh, effo, cmh, cmo,
            ob1_v, ob2_v, ob3_v, ob4_v, obs_v, obl_v,
            sem):
    cid = lax.axis_index("c")
    sid = lax.axis_index("s")

    @pl.when(jnp.logical_and(cid == 0, sid == 0))
    def _work():
        iota = _iota16()

        copies = [
            pltpu.async_copy(src, dst, sem)
            for src, dst in ((x1_h, x1_v), (y1_h, y1_v), (x2_h, x2_v),
                             (y2_h, y2_v), (sc_h, sc_v), (lb_h, lb_v))
        ]
        for c in copies:
            c.wait()

        def prep(j, mv):
            base = j * 16
            x1c = x1_v[pl.ds(base, 16)]
            y1c = y1_v[pl.ds(base, 16)]
            x2c = x2_v[pl.ds(base, 16)]
            y2c = y2_v[pl.ds(base, 16)]
            mv = jnp.maximum(mv, jnp.maximum(jnp.maximum(x1c, y1c),
                                             jnp.maximum(x2c, y2c)))
            scc = sc_v[pl.ds(base, 16)]
            lbc = lb_v[pl.ds(base, 16)]
            valid = scc >= _SCORE_THRESH
            ish = lbc == 1
            eh = jnp.where(jnp.logical_and(valid, ish), scc, _NEGF)
            eo = jnp.where(jnp.logical_and(valid, jnp.logical_not(ish)),
                           scc, _NEGF)
            effh[pl.ds(base, 16)] = eh
            effo[pl.ds(base, 16)] = eo
            cb = (j // 16) * 16
            mh = jnp.max(eh)
            mo = jnp.max(eo)
            lane = j - cb
            cmh[pl.ds(cb, 16)] = jnp.where(iota == lane, mh, cmh[pl.ds(cb, 16)])
            cmo[pl.ds(cb, 16)] = jnp.where(iota == lane, mo, cmo[pl.ds(cb, 16)])
            return mv

        neg16 = jnp.full((16,), _NEGF, _f32)
        for j in range(_CMVECS):
            cmh[pl.ds(j * 16, 16)] = neg16
            cmo[pl.ds(j * 16, 16)] = neg16
        mv = lax.fori_loop(0, _NCHUNK, prep, neg16)
        maxc = jnp.max(mv) + _f32(1.0)

        def run_stream(eff_ref, cm_ref):
            def cond(st):
                return jnp.logical_and(st[0] < _MAXK, jnp.logical_not(st[1]))

            def body(st):
                count, done, kx1, ky1, kx2, ky2, kar, selS, selI = st

                def scan(j, c):
                    mvv, ivv = c
                    v = cm_ref[pl.ds(j * 16, 16)]
                    idx = j * 16 + iota
                    gt = v > mvv
                    return (jnp.where(gt, v, mvv), jnp.where(gt, idx, ivv))

                mvv, ivv = lax.fori_loop(
                    0, _CMVECS, scan,
                    (jnp.full((16,), _NEGF, _f32), jnp.zeros((16,), _i32)))
                m = jnp.max(mvv)
                valid_m = m > _f32(-1e38)
                chunk = jnp.min(jnp.where(mvv == m, ivv, _i32(1 << 30)))
                ev = eff_ref[pl.ds(chunk * 16, 16)]
                lane = jnp.min(jnp.where(ev == m, iota, _i32(15)))
                i = chunk * 16 + lane

                cbase = chunk * 16
                lm = iota == lane

                def extf(ref):
                    return jnp.sum(jnp.where(lm, ref[pl.ds(cbase, 16)],
                                             _f32(0.0)))

                lbl = jnp.sum(jnp.where(lm, lb_v[pl.ds(cbase, 16)], _i32(0)))
                lblf = lbl.astype(_f32)
                off = lblf * maxc
                cx1 = extf(x1_v) + off
                cy1 = extf(y1_v) + off
                cx2 = extf(x2_v) + off
                cy2 = extf(y2_v) + off
                carea = (cx2 - cx1) * (cy2 - cy1)

                ltx = jnp.maximum(kx1, cx1)
                lty = jnp.maximum(ky1, cy1)
                rbx = jnp.minimum(kx2, cx2)
                rby = jnp.minimum(ky2, cy2)
                w = jnp.maximum(rbx - ltx, _f32(0.0))
                h = jnp.maximum(rby - lty, _f32(0.0))
                inter = w * h
                union = kar + carea - inter
                iou = inter / jnp.maximum(union, _f32(1e-9))
                supp = jnp.any(jnp.logical_and(iou > _NMS_THRESH, iota < count))
                keep = jnp.logical_and(jnp.logical_not(supp), valid_m)

                sel = jnp.logical_and(iota == count, keep)
                kx1 = jnp.where(sel, cx1, kx1)
                ky1 = jnp.where(sel, cy1, ky1)
                kx2 = jnp.where(sel, cx2, kx2)
                ky2 = jnp.where(sel, cy2, ky2)
                kar = jnp.where(sel, carea, kar)
                selS = jnp.where(sel, m, selS)
                selI = jnp.where(sel, i, selI)
                count = count + keep.astype(_i32)

                ev2 = jnp.where(iota == lane, _NEGF, ev)
                eff_ref[pl.ds(chunk * 16, 16)] = ev2
                newmax = jnp.max(ev2)
                cb = (chunk // 16) * 16
                cv = cm_ref[pl.ds(cb, 16)]
                cm_ref[pl.ds(cb, 16)] = jnp.where(iota == chunk - cb,
                                                  newmax, cv)
                done = jnp.logical_not(valid_m)
                return (count, done, kx1, ky1, kx2, ky2, kar, selS, selI)

            z16 = jnp.zeros((16,), _f32)
            st = lax.while_loop(
                cond, body,
                (_i32(0), False, z16, z16, z16, z16, z16,
                 jnp.full((16,), _NEGF, _f32), jnp.zeros((16,), _i32)))
            return st[7], st[8]

        hS, hI = run_stream(effh, cmh)
        oS, oI = run_stream(effo, cmo)

        z16 = jnp.zeros((16,), _f32)
        for j in range(2):
            ob1_v[pl.ds(j * 16, 16)] = z16
            ob2_v[pl.ds(j * 16, 16)] = z16
            ob3_v[pl.ds(j * 16, 16)] = z16
            ob4_v[pl.ds(j * 16, 16)] = z16
            obs_v[pl.ds(j * 16, 16)] = z16
            obl_v[pl.ds(j * 16, 16)] = jnp.full((16,), -1, _i32)

        def fext(vec, p):
            return jnp.sum(jnp.where(iota == p, vec, _f32(0.0)))

        def iext(vec, p):
            return jnp.sum(jnp.where(iota == p, vec, _i32(0)))

        def mbody(k, c):
            a, b = c
            ha = fext(hS, a)
            hi_ = iext(hI, a)
            oa = fext(oS, b)
            oi_ = iext(oI, b)
            take_h = jnp.logical_or(
                ha > oa, jnp.logical_and(ha == oa, hi_ < oi_))
            any_ = jnp.maximum(ha, oa) > _f32(-1e38)
            i = jnp.where(take_h, hi_, oi_)
            s = jnp.where(take_h, ha, oa)
            ibase = (i // 16) * 16
            ilm = iota == i - ibase

            def gext(ref):
                return jnp.sum(jnp.where(ilm, ref[pl.ds(ibase, 16)],
                                         _f32(0.0)))

            bx1 = jnp.where(any_, gext(x1_v), _f32(0.0))
            by1 = jnp.where(any_, gext(y1_v), _f32(0.0))
            bx2 = jnp.where(any_, gext(x2_v), _f32(0.0))
            by2 = jnp.where(any_, gext(y2_v), _f32(0.0))
            ssc = jnp.where(any_, s, _f32(0.0))
            ilbl = jnp.sum(jnp.where(ilm, lb_v[pl.ds(ibase, 16)], _i32(0)))
            slb = jnp.where(any_, ilbl, _i32(-1))
            kb = (k // 16) * 16
            kl = k - kb
            ob1_v[pl.ds(kb, 16)] = jnp.where(iota == kl, bx1, ob1_v[pl.ds(kb, 16)])
            ob2_v[pl.ds(kb, 16)] = jnp.where(iota == kl, by1, ob2_v[pl.ds(kb, 16)])
            ob3_v[pl.ds(kb, 16)] = jnp.where(iota == kl, bx2, ob3_v[pl.ds(kb, 16)])
            ob4_v[pl.ds(kb, 16)] = jnp.where(iota == kl, by2, ob4_v[pl.ds(kb, 16)])
            obs_v[pl.ds(kb, 16)] = jnp.where(iota == kl, ssc, obs_v[pl.ds(kb, 16)])
            obl_v[pl.ds(kb, 16)] = jnp.where(iota == kl, slb, obl_v[pl.ds(kb, 16)])
            taken = any_.astype(_i32)
            a = a + jnp.where(take_h, taken, 0)
            b = b + jnp.where(take_h, 0, taken)
            return (a, b)

        lax.fori_loop(0, 30, mbody, (_i32(0), _i32(0)))

        outs = [
            pltpu.async_copy(src, dst, sem)
            for src, dst in ((ob1_v, ox1_h), (ob2_v, oy1_h), (ob3_v, ox2_h),
                             (ob4_v, oy2_h), (obs_v, osc_h), (obl_v, olb_h))
        ]
        for c in outs:
            c.wait()


def kernel(boxes, scores, labels):
    pad = _NPAD - _N
    x1 = jnp.pad(boxes[:, 0], (0, pad))
    y1 = jnp.pad(boxes[:, 1], (0, pad))
    x2 = jnp.pad(boxes[:, 2], (0, pad))
    y2 = jnp.pad(boxes[:, 3], (0, pad))
    sc = jnp.pad(scores, (0, pad), constant_values=-1.0)
    lb = jnp.pad(labels, (0, pad))
    bx1, by1, bx2, by2, osc, olb = _nms_sc(x1, y1, x2, y2, sc, lb)
    out_boxes = jnp.stack([bx1, by1, bx2, by2], axis=1)[:30]
    return out_boxes, osc[:30], olb[:30]


# speedup vs baseline: 694.9058x; 1.1788x over previous
import functools

import jax
import jax.numpy as jnp
import numpy as np
from jax import lax
from jax.experimental import pallas as pl
from jax.experimental.pallas import tpu as pltpu
from jax.experimental.pallas import tpu_sc as plsc

_N = 5000
_NPAD = 5120
_NGROUP = 20
_NEGF = np.float32(-np.inf)
_SCORE_THRESH = np.float32(0.2)
_NMS_THRESH = np.float32(0.5)
_MAXK = 15

_mesh = plsc.VectorSubcoreMesh(core_axis_name="c", subcore_axis_name="s")

_f32 = np.float32
_i32 = np.int32


def _iota16():
    return lax.broadcasted_iota(_i32, (16,), 0)


@functools.partial(
    pl.kernel,
    out_type=[jax.ShapeDtypeStruct((32,), _f32)] * 5
    + [jax.ShapeDtypeStruct((32,), _i32)],
    mesh=_mesh,
    compiler_params=pltpu.CompilerParams(needs_layout_passes=False),
    scratch_types=[
        pltpu.VMEM((_NPAD,), _f32),
        pltpu.VMEM((_NPAD,), _f32),
        pltpu.VMEM((_NPAD,), _f32),
        pltpu.VMEM((_NPAD,), _f32),
        pltpu.VMEM((_NPAD,), _f32),
        pltpu.VMEM((_NPAD,), _i32),
        pltpu.VMEM((_NPAD,), _f32),
        pltpu.VMEM((_NGROUP * 16,), _f32),
        pltpu.VMEM((16,), _f32),
        pltpu.VMEM((16,), _i32),
        pltpu.VMEM((16,), _f32),
        pltpu.VMEM((16,), _i32),
        pltpu.VMEM((32,), _f32),
        pltpu.VMEM((32,), _f32),
        pltpu.VMEM((32,), _f32),
        pltpu.VMEM((32,), _f32),
        pltpu.VMEM((32,), _f32),
        pltpu.VMEM((32,), _i32),
        pltpu.VMEM_SHARED((16,), _f32),
        pltpu.VMEM_SHARED((16,), _i32),
        pltpu.SemaphoreType.DMA,
    ],
)
def _nms_sc(x1_h, y1_h, x2_h, y2_h, sc_h, lb_h,
            ox1_h, oy1_h, ox2_h, oy2_h, osc_h, olb_h,
            x1_v, y1_v, x2_v, y2_v, sc_v, lb_v,
            eff, cm, sS_v, sI_v, hS_v, hI_v,
            ob1_v, ob2_v, ob3_v, ob4_v, obs_v, obl_v,
            shS, shI, sem):
    cid = lax.axis_index("c")
    sid = lax.axis_index("s")
    iota = _iota16()

    def stage_prep_stream(want_human):
        copies = [
            pltpu.async_copy(src, dst, sem)
            for src, dst in ((x1_h, x1_v), (y1_h, y1_v), (x2_h, x2_v),
                             (y2_h, y2_v), (sc_h, sc_v), (lb_h, lb_v))
        ]
        for c in copies:
            c.wait()

        def prep_group(g, mv):
            base0 = g * 256
            acc = jnp.full((16,), _NEGF, _f32)
            for t in range(16):
                base = base0 + t * 16
                x2c = x2_v[pl.ds(base, 16)]
                y2c = y2_v[pl.ds(base, 16)]
                mv = jnp.maximum(mv, jnp.maximum(x2c, y2c))
                scc = sc_v[pl.ds(base, 16)]
                lbc = lb_v[pl.ds(base, 16)]
                valid = scc >= _SCORE_THRESH
                ish = lbc == 1
                want = ish if want_human else jnp.logical_not(ish)
                e = jnp.where(jnp.logical_and(valid, want), scc, _NEGF)
                eff[pl.ds(base, 16)] = e
                acc = jnp.where(iota == t, jnp.max(e), acc)
            cm[pl.ds(g * 16, 16)] = acc
            return mv

        mv = lax.fori_loop(0, _NGROUP, prep_group,
                           jnp.full((16,), _NEGF, _f32))
        maxc = jnp.max(mv) + _f32(1.0)

        def cond(st):
            return jnp.logical_and(st[0] < _MAXK, jnp.logical_not(st[1]))

        def body(st):
            count, done, kx1, ky1, kx2, ky2, kar, selS, selI = st

            mvv = jnp.full((16,), _NEGF, _f32)
            ivv = jnp.zeros((16,), _i32)
            for j in range(_NGROUP):
                v = cm[pl.ds(j * 16, 16)]
                gt = v > mvv
                mvv = jnp.where(gt, v, mvv)
                ivv = jnp.where(gt, j * 16 + iota, ivv)
            m = jnp.max(mvv)
            valid_m = m > _f32(-1e38)
            chunk = jnp.min(jnp.where(mvv == m, ivv, _i32(1 << 30)))
            ev = eff[pl.ds(chunk * 16, 16)]
            lane = jnp.min(jnp.where(ev == m, iota, _i32(15)))
            i = chunk * 16 + lane

            cbase = chunk * 16
            lm = iota == lane

            def extf(ref):
                return jnp.sum(jnp.where(lm, ref[pl.ds(cbase, 16)],
                                         _f32(0.0)))

            lbl = jnp.sum(jnp.where(lm, lb_v[pl.ds(cbase, 16)], _i32(0)))
            lblf = lbl.astype(_f32)
            off = lblf * maxc
            cx1 = extf(x1_v) + off
            cy1 = extf(y1_v) + off
            cx2 = extf(x2_v) + off
            cy2 = extf(y2_v) + off
            carea = (cx2 - cx1) * (cy2 - cy1)

            ltx = jnp.maximum(kx1, cx1)
            lty = jnp.maximum(ky1, cy1)
            rbx = jnp.minimum(kx2, cx2)
            rby = jnp.minimum(ky2, cy2)
            w = jnp.maximum(rbx - ltx, _f32(0.0))
            h = jnp.maximum(rby - lty, _f32(0.0))
            inter = w * h
            union = kar + carea - inter
            iou = inter / jnp.maximum(union, _f32(1e-9))
            supp = jnp.any(jnp.logical_and(iou > _NMS_THRESH, iota < count))
            keep = jnp.logical_and(jnp.logical_not(supp), valid_m)

            sel = jnp.logical_and(iota == count, keep)
            kx1 = jnp.where(sel, cx1, kx1)
            ky1 = jnp.where(sel, cy1, ky1)
            kx2 = jnp.where(sel, cx2, kx2)
            ky2 = jnp.where(sel, cy2, ky2)
            kar = jnp.where(sel, carea, kar)
            selS = jnp.where(sel, m, selS)
            selI = jnp.where(sel, i, selI)
            count = count + keep.astype(_i32)

            ev2 = jnp.where(lm, _NEGF, ev)
            eff[pl.ds(cbase, 16)] = ev2
            newmax = jnp.max(ev2)
            cb = (chunk // 16) * 16
            cv = cm[pl.ds(cb, 16)]
            cm[pl.ds(cb, 16)] = jnp.where(iota == chunk - cb, newmax, cv)
            done = jnp.logical_not(valid_m)
            return (count, done, kx1, ky1, kx2, ky2, kar, selS, selI)

        z16 = jnp.zeros((16,), _f32)
        st = lax.while_loop(
            cond, body,
            (_i32(0), False, z16, z16, z16, z16, z16,
             jnp.full((16,), _NEGF, _f32), jnp.zeros((16,), _i32)))
        sS_v[pl.ds(0, 16)] = st[7]
        sI_v[pl.ds(0, 16)] = st[8]

    @pl.when(jnp.logical_and(cid == 0, sid == 0))
    def _object_stream():
        stage_prep_stream(want_human=False)

    @pl.when(jnp.logical_and(cid == 0, sid == 1))
    def _human_stream():
        stage_prep_stream(want_human=True)
        pltpu.sync_copy(sS_v, shS)
        pltpu.sync_copy(sI_v, shI)

    plsc.subcore_barrier()

    @pl.when(jnp.logical_and(cid == 0, sid == 0))
    def _merge():
        pltpu.sync_copy(shS, hS_v)
        pltpu.sync_copy(shI, hI_v)
        hS = hS_v[pl.ds(0, 16)]
        hI = hI_v[pl.ds(0, 16)]
        oS = sS_v[pl.ds(0, 16)]
        oI = sI_v[pl.ds(0, 16)]

        z16 = jnp.zeros((16,), _f32)
        for j in range(2):
            ob1_v[pl.ds(j * 16, 16)] = z16
            ob2_v[pl.ds(j * 16, 16)] = z16
            ob3_v[pl.ds(j * 16, 16)] = z16
            ob4_v[pl.ds(j * 16, 16)] = z16
            obs_v[pl.ds(j * 16, 16)] = z16
            obl_v[pl.ds(j * 16, 16)] = jnp.full((16,), -1, _i32)

        def fext(vec, p):
            return jnp.sum(jnp.where(iota == p, vec, _f32(0.0)))

        def iext(vec, p):
            return jnp.sum(jnp.where(iota == p, vec, _i32(0)))

        def mbody(k, c):
            a, b = c
            ha = fext(hS, a)
            hi_ = iext(hI, a)
            oa = fext(oS, b)
            oi_ = iext(oI, b)
            take_h = jnp.logical_or(
                ha > oa, jnp.logical_and(ha == oa, hi_ < oi_))
            any_ = jnp.maximum(ha, oa) > _f32(-1e38)
            i = jnp.where(take_h, hi_, oi_)
            s = jnp.where(take_h, ha, oa)
            ibase = (i // 16) * 16
            ilm = iota == i - ibase

            def gext(ref):
                return jnp.sum(jnp.where(ilm, ref[pl.ds(ibase, 16)],
                                         _f32(0.0)))

            bx1 = jnp.where(any_, gext(x1_v), _f32(0.0))
            by1 = jnp.where(any_, gext(y1_v), _f32(0.0))
            bx2 = jnp.where(any_, gext(x2_v), _f32(0.0))
            by2 = jnp.where(any_, gext(y2_v), _f32(0.0))
            ssc = jnp.where(any_, s, _f32(0.0))
            ilbl = jnp.sum(jnp.where(ilm, lb_v[pl.ds(ibase, 16)], _i32(0)))
            slb = jnp.where(any_, ilbl, _i32(-1))
            kb = (k // 16) * 16
            kl = k - kb
            km = iota == kl
            ob1_v[pl.ds(kb, 16)] = jnp.where(km, bx1, ob1_v[pl.ds(kb, 16)])
            ob2_v[pl.ds(kb, 16)] = jnp.where(km, by1, ob2_v[pl.ds(kb, 16)])
            ob3_v[pl.ds(kb, 16)] = jnp.where(km, bx2, ob3_v[pl.ds(kb, 16)])
            ob4_v[pl.ds(kb, 16)] = jnp.where(km, by2, ob4_v[pl.ds(kb, 16)])
            obs_v[pl.ds(kb, 16)] = jnp.where(km, ssc, obs_v[pl.ds(kb, 16)])
            obl_v[pl.ds(kb, 16)] = jnp.where(km, slb, obl_v[pl.ds(kb, 16)])
            taken = any_.astype(_i32)
            a = a + jnp.where(take_h, taken, 0)
            b = b + jnp.where(take_h, 0, taken)
            return (a, b)

        lax.fori_loop(0, 30, mbody, (_i32(0), _i32(0)))

        outs = [
            pltpu.async_copy(src, dst, sem)
            for src, dst in ((ob1_v, ox1_h), (ob2_v, oy1_h), (ob3_v, ox2_h),
                             (ob4_v, oy2_h), (obs_v, osc_h), (obl_v, olb_h))
        ]
        for c in outs:
            c.wait()


def kernel(boxes, scores, labels):
    pad = _NPAD - _N
    x1 = jnp.pad(boxes[:, 0], (0, pad))
    y1 = jnp.pad(boxes[:, 1], (0, pad))
    x2 = jnp.pad(boxes[:, 2], (0, pad))
    y2 = jnp.pad(boxes[:, 3], (0, pad))
    sc = jnp.pad(scores, (0, pad), constant_values=-1.0)
    lb = jnp.pad(labels, (0, pad))
    bx1, by1, bx2, by2, osc, olb = _nms_sc(x1, y1, x2, y2, sc, lb)
    out_boxes = jnp.stack([bx1, by1, bx2, by2], axis=1)[:30]
    return out_boxes, osc[:30], olb[:30]


# speedup vs baseline: 730.1065x; 1.0507x over previous
import functools

import jax
import jax.numpy as jnp
import numpy as np
from jax import lax
from jax.experimental import pallas as pl
from jax.experimental.pallas import tpu as pltpu
from jax.experimental.pallas import tpu_sc as plsc

_N = 5000
_NPAD = 5120
_NGROUP = 20
_NEGF = np.float32(-np.inf)
_SCORE_THRESH = np.float32(0.2)
_NMS_THRESH = np.float32(0.5)
_MAXK = 15

_mesh = plsc.VectorSubcoreMesh(core_axis_name="c", subcore_axis_name="s",
                               num_cores=1)

_f32 = np.float32
_i32 = np.int32


def _iota16():
    return lax.broadcasted_iota(_i32, (16,), 0)


@functools.partial(
    pl.kernel,
    out_type=[jax.ShapeDtypeStruct((32,), _f32)] * 5
    + [jax.ShapeDtypeStruct((32,), _i32)],
    mesh=_mesh,
    compiler_params=pltpu.CompilerParams(needs_layout_passes=False),
    scratch_types=[
        pltpu.VMEM((_NPAD,), _f32),
        pltpu.VMEM((_NPAD,), _f32),
        pltpu.VMEM((_NPAD,), _f32),
        pltpu.VMEM((_NPAD,), _f32),
        pltpu.VMEM((_NPAD,), _f32),
        pltpu.VMEM((_NPAD,), _i32),
        pltpu.VMEM((_NPAD,), _f32),
        pltpu.VMEM((_NGROUP * 16,), _f32),
        pltpu.VMEM((16,), _f32),
        pltpu.VMEM((16,), _i32),
        pltpu.VMEM((16,), _f32),
        pltpu.VMEM((16,), _i32),
        pltpu.VMEM((32,), _f32),
        pltpu.VMEM((32,), _f32),
        pltpu.VMEM((32,), _f32),
        pltpu.VMEM((32,), _f32),
        pltpu.VMEM((32,), _f32),
        pltpu.VMEM((32,), _i32),
        pltpu.VMEM_SHARED((16,), _f32),
        pltpu.VMEM_SHARED((16,), _i32),
        pltpu.SemaphoreType.DMA,
    ],
)
def _nms_sc(x1_h, y1_h, x2_h, y2_h, sc_h, lb_h,
            ox1_h, oy1_h, ox2_h, oy2_h, osc_h, olb_h,
            x1_v, y1_v, x2_v, y2_v, sc_v, lb_v,
            eff, cm, sS_v, sI_v, hS_v, hI_v,
            ob1_v, ob2_v, ob3_v, ob4_v, obs_v, obl_v,
            shS, shI, sem):
    cid = lax.axis_index("c")
    sid = lax.axis_index("s")
    iota = _iota16()

    def stage_prep_stream(want_human):
        copies = [
            pltpu.async_copy(src, dst, sem)
            for src, dst in ((x1_h, x1_v), (y1_h, y1_v), (x2_h, x2_v),
                             (y2_h, y2_v), (sc_h, sc_v), (lb_h, lb_v))
        ]
        for c in copies:
            c.wait()

        def prep_group(g, mv):
            base0 = g * 256
            acc = jnp.full((16,), _NEGF, _f32)
            for t in range(16):
                base = base0 + t * 16
                x2c = x2_v[pl.ds(base, 16)]
                y2c = y2_v[pl.ds(base, 16)]
                mv = jnp.maximum(mv, jnp.maximum(x2c, y2c))
                scc = sc_v[pl.ds(base, 16)]
                lbc = lb_v[pl.ds(base, 16)]
                valid = scc >= _SCORE_THRESH
                ish = lbc == 1
                want = ish if want_human else jnp.logical_not(ish)
                e = jnp.where(jnp.logical_and(valid, want), scc, _NEGF)
                eff[pl.ds(base, 16)] = e
                acc = jnp.where(iota == t, jnp.max(e), acc)
            cm[pl.ds(g * 16, 16)] = acc
            return mv

        mv = lax.fori_loop(0, _NGROUP, prep_group,
                           jnp.full((16,), _NEGF, _f32))
        maxc = jnp.max(mv) + _f32(1.0)

        def cond(st):
            return jnp.logical_and(st[0] < _MAXK, jnp.logical_not(st[1]))

        def body(st):
            count, done, kx1, ky1, kx2, ky2, kar, selS, selI = st

            mvv = jnp.full((16,), _NEGF, _f32)
            ivv = jnp.zeros((16,), _i32)
            for j in range(_NGROUP):
                v = cm[pl.ds(j * 16, 16)]
                gt = v > mvv
                mvv = jnp.where(gt, v, mvv)
                ivv = jnp.where(gt, j * 16 + iota, ivv)
            m = jnp.max(mvv)
            valid_m = m > _f32(-1e38)
            chunk = jnp.min(jnp.where(mvv == m, ivv, _i32(1 << 30)))
            ev = eff[pl.ds(chunk * 16, 16)]
            lane = jnp.min(jnp.where(ev == m, iota, _i32(15)))
            i = chunk * 16 + lane

            cbase = chunk * 16
            lm = iota == lane

            def extf(ref):
                return jnp.sum(jnp.where(lm, ref[pl.ds(cbase, 16)],
                                         _f32(0.0)))

            lbl = jnp.sum(jnp.where(lm, lb_v[pl.ds(cbase, 16)], _i32(0)))
            lblf = lbl.astype(_f32)
            off = lblf * maxc
            cx1 = extf(x1_v) + off
            cy1 = extf(y1_v) + off
            cx2 = extf(x2_v) + off
            cy2 = extf(y2_v) + off
            carea = (cx2 - cx1) * (cy2 - cy1)

            ltx = jnp.maximum(kx1, cx1)
            lty = jnp.maximum(ky1, cy1)
            rbx = jnp.minimum(kx2, cx2)
            rby = jnp.minimum(ky2, cy2)
            w = jnp.maximum(rbx - ltx, _f32(0.0))
            h = jnp.maximum(rby - lty, _f32(0.0))
            inter = w * h
            union = kar + carea - inter
            iou = inter / jnp.maximum(union, _f32(1e-9))
            supp = jnp.any(jnp.logical_and(iou > _NMS_THRESH, iota < count))
            keep = jnp.logical_and(jnp.logical_not(supp), valid_m)

            sel = jnp.logical_and(iota == count, keep)
            kx1 = jnp.where(sel, cx1, kx1)
            ky1 = jnp.where(sel, cy1, ky1)
            kx2 = jnp.where(sel, cx2, kx2)
            ky2 = jnp.where(sel, cy2, ky2)
            kar = jnp.where(sel, carea, kar)
            selS = jnp.where(sel, m, selS)
            selI = jnp.where(sel, i, selI)
            count = count + keep.astype(_i32)

            ev2 = jnp.where(lm, _NEGF, ev)
            eff[pl.ds(cbase, 16)] = ev2
            newmax = jnp.max(ev2)
            cb = (chunk // 16) * 16
            cv = cm[pl.ds(cb, 16)]
            cm[pl.ds(cb, 16)] = jnp.where(iota == chunk - cb, newmax, cv)
            done = jnp.logical_not(valid_m)
            return (count, done, kx1, ky1, kx2, ky2, kar, selS, selI)

        z16 = jnp.zeros((16,), _f32)
        st = lax.while_loop(
            cond, body,
            (_i32(0), False, z16, z16, z16, z16, z16,
             jnp.full((16,), _NEGF, _f32), jnp.zeros((16,), _i32)))
        sS_v[pl.ds(0, 16)] = st[7]
        sI_v[pl.ds(0, 16)] = st[8]

    @pl.when(jnp.logical_and(cid == 0, sid == 0))
    def _object_stream():
        stage_prep_stream(want_human=False)

    @pl.when(jnp.logical_and(cid == 0, sid == 1))
    def _human_stream():
        stage_prep_stream(want_human=True)
        pltpu.sync_copy(sS_v, shS)
        pltpu.sync_copy(sI_v, shI)

    plsc.subcore_barrier()

    @pl.when(jnp.logical_and(cid == 0, sid == 0))
    def _merge():
        pltpu.sync_copy(shS, hS_v)
        pltpu.sync_copy(shI, hI_v)
        hS = hS_v[pl.ds(0, 16)]
        hI = hI_v[pl.ds(0, 16)]
        oS = sS_v[pl.ds(0, 16)]
        oI = sI_v[pl.ds(0, 16)]

        z16 = jnp.zeros((16,), _f32)
        for j in range(2):
            ob1_v[pl.ds(j * 16, 16)] = z16
            ob2_v[pl.ds(j * 16, 16)] = z16
            ob3_v[pl.ds(j * 16, 16)] = z16
            ob4_v[pl.ds(j * 16, 16)] = z16
            obs_v[pl.ds(j * 16, 16)] = z16
            obl_v[pl.ds(j * 16, 16)] = jnp.full((16,), -1, _i32)

        def fext(vec, p):
            return jnp.sum(jnp.where(iota == p, vec, _f32(0.0)))

        def iext(vec, p):
            return jnp.sum(jnp.where(iota == p, vec, _i32(0)))

        def mbody(k, c):
            a, b = c
            ha = fext(hS, a)
            hi_ = iext(hI, a)
            oa = fext(oS, b)
            oi_ = iext(oI, b)
            take_h = jnp.logical_or(
                ha > oa, jnp.logical_and(ha == oa, hi_ < oi_))
            any_ = jnp.maximum(ha, oa) > _f32(-1e38)
            i = jnp.where(take_h, hi_, oi_)
            s = jnp.where(take_h, ha, oa)
            ibase = (i // 16) * 16
            ilm = iota == i - ibase

            def gext(ref):
                return jnp.sum(jnp.where(ilm, ref[pl.ds(ibase, 16)],
                                         _f32(0.0)))

            bx1 = jnp.where(any_, gext(x1_v), _f32(0.0))
            by1 = jnp.where(any_, gext(y1_v), _f32(0.0))
            bx2 = jnp.where(any_, gext(x2_v), _f32(0.0))
            by2 = jnp.where(any_, gext(y2_v), _f32(0.0))
            ssc = jnp.where(any_, s, _f32(0.0))
            ilbl = jnp.sum(jnp.where(ilm, lb_v[pl.ds(ibase, 16)], _i32(0)))
            slb = jnp.where(any_, ilbl, _i32(-1))
            kb = (k // 16) * 16
            kl = k - kb
            km = iota == kl
            ob1_v[pl.ds(kb, 16)] = jnp.where(km, bx1, ob1_v[pl.ds(kb, 16)])
            ob2_v[pl.ds(kb, 16)] = jnp.where(km, by1, ob2_v[pl.ds(kb, 16)])
            ob3_v[pl.ds(kb, 16)] = jnp.where(km, bx2, ob3_v[pl.ds(kb, 16)])
            ob4_v[pl.ds(kb, 16)] = jnp.where(km, by2, ob4_v[pl.ds(kb, 16)])
            obs_v[pl.ds(kb, 16)] = jnp.where(km, ssc, obs_v[pl.ds(kb, 16)])
            obl_v[pl.ds(kb, 16)] = jnp.where(km, slb, obl_v[pl.ds(kb, 16)])
            taken = any_.astype(_i32)
            a = a + jnp.where(take_h, taken, 0)
            b = b + jnp.where(take_h, 0, taken)
            return (a, b)

        lax.fori_loop(0, 30, mbody, (_i32(0), _i32(0)))

        outs = [
            pltpu.async_copy(src, dst, sem)
            for src, dst in ((ob1_v, ox1_h), (ob2_v, oy1_h), (ob3_v, ox2_h),
                             (ob4_v, oy2_h), (obs_v, osc_h), (obl_v, olb_h))
        ]
        for c in outs:
            c.wait()


def kernel(boxes, scores, labels):
    pad = _NPAD - _N
    x1 = jnp.pad(boxes[:, 0], (0, pad))
    y1 = jnp.pad(boxes[:, 1], (0, pad))
    x2 = jnp.pad(boxes[:, 2], (0, pad))
    y2 = jnp.pad(boxes[:, 3], (0, pad))
    sc = jnp.pad(scores, (0, pad), constant_values=-1.0)
    lb = jnp.pad(labels, (0, pad))
    bx1, by1, bx2, by2, osc, olb = _nms_sc(x1, y1, x2, y2, sc, lb)
    out_boxes = jnp.stack([bx1, by1, bx2, by2], axis=1)[:30]
    return out_boxes, osc[:30], olb[:30]
